# baseline (speedup 1.0000x reference)
import functools

import jax
import jax.numpy as jnp
from jax.experimental import pallas as pl
from jax.experimental.pallas import tpu as pltpu

B, C, H, W = 32, 384, 56, 56
E = 16
K = 2
BLKC = 128


def _stats_kernel(x_ref, out_ref):
    x = x_ref[...].reshape(BLKC, H, W)
    r = jax.lax.broadcasted_iota(jnp.int32, (W, W), 0)
    c = jax.lax.broadcasted_iota(jnp.int32, (W, W), 1)
    d = r - c
    tri = (jnp.where(d == 0, 2.0, 0.0)
           - jnp.where(d == 1, 1.0, 0.0)
           - jnp.where(d == -1, 1.0, 0.0)).astype(jnp.float32)
    hpart = jax.lax.dot_general(
        x.reshape(BLKC * H, W), tri,
        (((1,), (0,)), ((), ())),
        preferred_element_type=jnp.float32,
    ).reshape(BLKC, H, W)
    z = jnp.zeros((BLKC, 1, W), dtype=x.dtype)
    up = jnp.concatenate([z, x[:, :-1, :]], axis=1)
    down = jnp.concatenate([x[:, 1:, :], z], axis=1)
    lap = (2.0 * x - up - down) + hpart
    inv = jnp.float32(1.0 / (H * W))
    s_mean = jnp.sum(x, axis=(1, 2)) * inv
    s_freq = jnp.sum(jnp.abs(lap), axis=(1, 2)) * inv
    out_ref[...] = jnp.stack([s_mean, s_freq], axis=1).reshape(1, BLKC, 2)


def _router_kernel(c_ref, w1_ref, b1_ref, w2_ref, b2_ref, rw_ref, idx_ref):
    combined = c_ref[...]
    h1 = jax.lax.dot_general(
        combined, w1_ref[...],
        (((1,), (1,)), ((), ())),
        preferred_element_type=jnp.float32,
    ) + b1_ref[...]
    h1 = jnp.maximum(h1, 0.0)
    logits = jax.lax.dot_general(
        h1, w2_ref[...],
        (((1,), (1,)), ((), ())),
        preferred_element_type=jnp.float32,
    ) + b2_ref[...]

    iota = jax.lax.broadcasted_iota(jnp.int32, (B, E), 1)
    m1 = jnp.max(logits, axis=1, keepdims=True)
    i1 = jnp.min(jnp.where(logits == m1, iota, E), axis=1, keepdims=True)
    neg = jnp.float32(-3.0e38)
    masked = jnp.where(iota == i1, neg, logits)
    m2 = jnp.max(masked, axis=1, keepdims=True)
    i2 = jnp.min(jnp.where(masked == m2, iota, E), axis=1, keepdims=True)

    e2 = jnp.exp(m2 - m1)
    denom = 1.0 + e2
    p1 = 1.0 / denom
    p2 = e2 / denom

    rw_ref[...] = jnp.where(iota == i1, p1, 0.0) + jnp.where(iota == i2, p2, 0.0)
    idx_ref[...] = jnp.concatenate([i1, i2], axis=1)


@functools.partial(jax.jit, static_argnames=("interpret",))
def _run(x, W1, b1, W2, b2, interpret=False):
    stats = pl.pallas_call(
        _stats_kernel,
        grid=(B, C // BLKC),
        in_specs=[pl.BlockSpec((1, BLKC, H, W), lambda i, j: (i, j, 0, 0))],
        out_specs=pl.BlockSpec((1, BLKC, 2), lambda i, j: (i, j, 0)),
        out_shape=jax.ShapeDtypeStruct((B, C, 2), jnp.float32),
        compiler_params=pltpu.CompilerParams(
            dimension_semantics=("parallel", "parallel"),
        ),
        interpret=interpret,
    )(x)
    combined = jnp.concatenate([stats[:, :, 0], stats[:, :, 1]], axis=1)

    rw, idx = pl.pallas_call(
        _router_kernel,
        in_specs=[
            pl.BlockSpec((B, 2 * C), lambda: (0, 0)),
            pl.BlockSpec((C, 2 * C), lambda: (0, 0)),
            pl.BlockSpec((1, C), lambda: (0, 0)),
            pl.BlockSpec((E, C), lambda: (0, 0)),
            pl.BlockSpec((1, E), lambda: (0, 0)),
        ],
        out_specs=[
            pl.BlockSpec((B, E), lambda: (0, 0)),
            pl.BlockSpec((B, K), lambda: (0, 0)),
        ],
        out_shape=[
            jax.ShapeDtypeStruct((B, E), jnp.float32),
            jax.ShapeDtypeStruct((B, K), jnp.int32),
        ],
        interpret=interpret,
    )(combined, W1, b1.reshape(1, C), W2, b2.reshape(1, E))
    return rw.reshape(B, E, 1, 1), idx.reshape(B, K, 1, 1)


def kernel(x, W1, b1, W2, b2):
    return _run(x, W1, b1, W2, b2)


# speedup vs baseline: 4.1354x; 4.1354x over previous
import functools

import jax
import jax.numpy as jnp
from jax.experimental import pallas as pl
from jax.experimental.pallas import tpu as pltpu

B, C, H, W = 32, 384, 56, 56
E = 16
K = 2


def _stats_kernel(x_ref, out_ref):
    x = x_ref[0]
    zr = jnp.zeros((1, W, C), dtype=x.dtype)
    zc = jnp.zeros((H, 1, C), dtype=x.dtype)
    up = jnp.concatenate([zr, x[:-1]], axis=0)
    down = jnp.concatenate([x[1:], zr], axis=0)
    left = jnp.concatenate([zc, x[:, :-1]], axis=1)
    right = jnp.concatenate([x[:, 1:], zc], axis=1)
    lap = 4.0 * x - up - down - left - right
    inv = jnp.float32(1.0 / (H * W))
    s_mean = jnp.sum(x, axis=(0, 1)) * inv
    s_freq = jnp.sum(jnp.abs(lap), axis=(0, 1)) * inv
    out_ref[0] = jnp.stack([s_mean, s_freq], axis=0)


def _router_kernel(c_ref, w1_ref, b1_ref, w2_ref, b2_ref, rw_ref, idx_ref):
    combined = c_ref[...]
    h1 = jax.lax.dot_general(
        combined, w1_ref[...],
        (((1,), (1,)), ((), ())),
        preferred_element_type=jnp.float32,
    ) + b1_ref[...]
    h1 = jnp.maximum(h1, 0.0)
    logits = jax.lax.dot_general(
        h1, w2_ref[...],
        (((1,), (1,)), ((), ())),
        preferred_element_type=jnp.float32,
    ) + b2_ref[...]

    iota = jax.lax.broadcasted_iota(jnp.int32, (B, E), 1)
    m1 = jnp.max(logits, axis=1, keepdims=True)
    i1 = jnp.min(jnp.where(logits == m1, iota, E), axis=1, keepdims=True)
    neg = jnp.float32(-3.0e38)
    masked = jnp.where(iota == i1, neg, logits)
    m2 = jnp.max(masked, axis=1, keepdims=True)
    i2 = jnp.min(jnp.where(masked == m2, iota, E), axis=1, keepdims=True)

    e2 = jnp.exp(m2 - m1)
    denom = 1.0 + e2
    p1 = 1.0 / denom
    p2 = e2 / denom

    rw_ref[...] = jnp.where(iota == i1, p1, 0.0) + jnp.where(iota == i2, p2, 0.0)
    idx_ref[...] = jnp.concatenate([i1, i2], axis=1)


@functools.partial(jax.jit, static_argnames=("interpret",))
def _run(x, W1, b1, W2, b2, interpret=False):
    xt = jnp.transpose(x, (0, 2, 3, 1))
    stats = pl.pallas_call(
        _stats_kernel,
        grid=(B,),
        in_specs=[pl.BlockSpec((1, H, W, C), lambda i: (i, 0, 0, 0))],
        out_specs=pl.BlockSpec((1, 2, C), lambda i: (i, 0, 0)),
        out_shape=jax.ShapeDtypeStruct((B, 2, C), jnp.float32),
        compiler_params=pltpu.CompilerParams(
            dimension_semantics=("arbitrary",),
        ),
        interpret=interpret,
    )(xt)
    combined = stats.reshape(B, 2 * C)

    rw, idx = pl.pallas_call(
        _router_kernel,
        in_specs=[
            pl.BlockSpec((B, 2 * C), lambda: (0, 0)),
            pl.BlockSpec((C, 2 * C), lambda: (0, 0)),
            pl.BlockSpec((1, C), lambda: (0, 0)),
            pl.BlockSpec((E, C), lambda: (0, 0)),
            pl.BlockSpec((1, E), lambda: (0, 0)),
        ],
        out_specs=[
            pl.BlockSpec((B, E), lambda: (0, 0)),
            pl.BlockSpec((B, K), lambda: (0, 0)),
        ],
        out_shape=[
            jax.ShapeDtypeStruct((B, E), jnp.float32),
            jax.ShapeDtypeStruct((B, K), jnp.int32),
        ],
        interpret=interpret,
    )(combined, W1, b1.reshape(1, C), W2, b2.reshape(1, E))
    return rw.reshape(B, E, 1, 1), idx.reshape(B, K, 1, 1)


def kernel(x, W1, b1, W2, b2):
    return _run(x, W1, b1, W2, b2)


# speedup vs baseline: 5.4321x; 1.3136x over previous
import functools

import jax
import jax.numpy as jnp
from jax.experimental import pallas as pl
from jax.experimental.pallas import tpu as pltpu

B, C, H, W = 32, 384, 56, 56
E = 16
K = 2


def _stats_kernel(x_ref, out_ref):
    zc = jnp.zeros((1, C), dtype=jnp.float32)
    accm = jnp.zeros((W, C), dtype=jnp.float32)
    accf = jnp.zeros((W, C), dtype=jnp.float32)
    prev = None
    cur = x_ref[0, 0]
    for h in range(H):
        nxt = x_ref[0, h + 1] if h + 1 < H else None
        lft = jnp.concatenate([zc, cur[:-1]], axis=0)
        rgt = jnp.concatenate([cur[1:], zc], axis=0)
        lap = 4.0 * cur - lft - rgt
        if prev is not None:
            lap = lap - prev
        if nxt is not None:
            lap = lap - nxt
        accm = accm + cur
        accf = accf + jnp.abs(lap)
        prev, cur = cur, nxt
    inv = jnp.float32(1.0 / (H * W))
    s_mean = jnp.sum(accm, axis=0) * inv
    s_freq = jnp.sum(accf, axis=0) * inv
    out_ref[0] = jnp.stack([s_mean, s_freq], axis=0)


def _router_kernel(c_ref, w1_ref, b1_ref, w2_ref, b2_ref, rw_ref, idx_ref):
    combined = c_ref[...]
    h1 = jax.lax.dot_general(
        combined, w1_ref[...],
        (((1,), (1,)), ((), ())),
        preferred_element_type=jnp.float32,
    ) + b1_ref[...]
    h1 = jnp.maximum(h1, 0.0)
    logits = jax.lax.dot_general(
        h1, w2_ref[...],
        (((1,), (1,)), ((), ())),
        preferred_element_type=jnp.float32,
    ) + b2_ref[...]

    iota = jax.lax.broadcasted_iota(jnp.int32, (B, E), 1)
    m1 = jnp.max(logits, axis=1, keepdims=True)
    i1 = jnp.min(jnp.where(logits == m1, iota, E), axis=1, keepdims=True)
    neg = jnp.float32(-3.0e38)
    masked = jnp.where(iota == i1, neg, logits)
    m2 = jnp.max(masked, axis=1, keepdims=True)
    i2 = jnp.min(jnp.where(masked == m2, iota, E), axis=1, keepdims=True)

    e2 = jnp.exp(m2 - m1)
    denom = 1.0 + e2
    p1 = 1.0 / denom
    p2 = e2 / denom

    rw_ref[...] = jnp.where(iota == i1, p1, 0.0) + jnp.where(iota == i2, p2, 0.0)
    idx_ref[...] = jnp.concatenate([i1, i2], axis=1)


@functools.partial(jax.jit, static_argnames=("interpret",))
def _run(x, W1, b1, W2, b2, interpret=False):
    xt = jnp.transpose(x, (0, 2, 3, 1))
    stats = pl.pallas_call(
        _stats_kernel,
        grid=(B,),
        in_specs=[pl.BlockSpec((1, H, W, C), lambda i: (i, 0, 0, 0))],
        out_specs=pl.BlockSpec((1, 2, C), lambda i: (i, 0, 0)),
        out_shape=jax.ShapeDtypeStruct((B, 2, C), jnp.float32),
        compiler_params=pltpu.CompilerParams(
            dimension_semantics=("arbitrary",),
        ),
        interpret=interpret,
    )(xt)
    combined = stats.reshape(B, 2 * C)

    rw, idx = pl.pallas_call(
        _router_kernel,
        in_specs=[
            pl.BlockSpec((B, 2 * C), lambda: (0, 0)),
            pl.BlockSpec((C, 2 * C), lambda: (0, 0)),
            pl.BlockSpec((1, C), lambda: (0, 0)),
            pl.BlockSpec((E, C), lambda: (0, 0)),
            pl.BlockSpec((1, E), lambda: (0, 0)),
        ],
        out_specs=[
            pl.BlockSpec((B, E), lambda: (0, 0)),
            pl.BlockSpec((B, K), lambda: (0, 0)),
        ],
        out_shape=[
            jax.ShapeDtypeStruct((B, E), jnp.float32),
            jax.ShapeDtypeStruct((B, K), jnp.int32),
        ],
        interpret=interpret,
    )(combined, W1, b1.reshape(1, C), W2, b2.reshape(1, E))
    return rw.reshape(B, E, 1, 1), idx.reshape(B, K, 1, 1)


def kernel(x, W1, b1, W2, b2):
    return _run(x, W1, b1, W2, b2)


# speedup vs baseline: 5.9718x; 1.0994x over previous
import functools

import jax
import jax.numpy as jnp
from jax.experimental import pallas as pl
from jax.experimental.pallas import tpu as pltpu

B, C, H, W = 32, 384, 56, 56
E = 16
K = 2


def _stats_kernel(x_ref, out_ref):
    r = jax.lax.broadcasted_iota(jnp.int32, (W, W), 0)
    c = jax.lax.broadcasted_iota(jnp.int32, (W, W), 1)
    nb = (jnp.abs(r - c) == 1).astype(jnp.float32)
    accm = jnp.zeros((W, C), dtype=jnp.float32)
    accf = jnp.zeros((W, C), dtype=jnp.float32)
    for h in range(H):
        cur = x_ref[0, h]
        wsum = jax.lax.dot_general(
            nb, cur,
            (((1,), (0,)), ((), ())),
            preferred_element_type=jnp.float32,
        )
        lap = 4.0 * cur - wsum
        if h > 0:
            lap = lap - x_ref[0, h - 1]
        if h + 1 < H:
            lap = lap - x_ref[0, h + 1]
        accm = accm + cur
        accf = accf + jnp.abs(lap)
    inv = jnp.float32(1.0 / (H * W))
    s_mean = jnp.sum(accm, axis=0) * inv
    s_freq = jnp.sum(accf, axis=0) * inv
    out_ref[0] = jnp.stack([s_mean, s_freq], axis=0)


def _router_kernel(c_ref, w1_ref, b1_ref, w2_ref, b2_ref, rw_ref, idx_ref):
    combined = c_ref[...]
    h1 = jax.lax.dot_general(
        combined, w1_ref[...],
        (((1,), (1,)), ((), ())),
        preferred_element_type=jnp.float32,
    ) + b1_ref[...]
    h1 = jnp.maximum(h1, 0.0)
    logits = jax.lax.dot_general(
        h1, w2_ref[...],
        (((1,), (1,)), ((), ())),
        preferred_element_type=jnp.float32,
    ) + b2_ref[...]

    iota = jax.lax.broadcasted_iota(jnp.int32, (B, E), 1)
    m1 = jnp.max(logits, axis=1, keepdims=True)
    i1 = jnp.min(jnp.where(logits == m1, iota, E), axis=1, keepdims=True)
    neg = jnp.float32(-3.0e38)
    masked = jnp.where(iota == i1, neg, logits)
    m2 = jnp.max(masked, axis=1, keepdims=True)
    i2 = jnp.min(jnp.where(masked == m2, iota, E), axis=1, keepdims=True)

    e2 = jnp.exp(m2 - m1)
    denom = 1.0 + e2
    p1 = 1.0 / denom
    p2 = e2 / denom

    rw_ref[...] = jnp.where(iota == i1, p1, 0.0) + jnp.where(iota == i2, p2, 0.0)
    idx_ref[...] = jnp.concatenate([i1, i2], axis=1)


@functools.partial(jax.jit, static_argnames=("interpret",))
def _run(x, W1, b1, W2, b2, interpret=False):
    xt = jnp.transpose(x, (0, 2, 3, 1))
    stats = pl.pallas_call(
        _stats_kernel,
        grid=(B,),
        in_specs=[pl.BlockSpec((1, H, W, C), lambda i: (i, 0, 0, 0))],
        out_specs=pl.BlockSpec((1, 2, C), lambda i: (i, 0, 0)),
        out_shape=jax.ShapeDtypeStruct((B, 2, C), jnp.float32),
        compiler_params=pltpu.CompilerParams(
            dimension_semantics=("arbitrary",),
        ),
        interpret=interpret,
    )(xt)
    combined = stats.reshape(B, 2 * C)

    rw, idx = pl.pallas_call(
        _router_kernel,
        in_specs=[
            pl.BlockSpec((B, 2 * C), lambda: (0, 0)),
            pl.BlockSpec((C, 2 * C), lambda: (0, 0)),
            pl.BlockSpec((1, C), lambda: (0, 0)),
            pl.BlockSpec((E, C), lambda: (0, 0)),
            pl.BlockSpec((1, E), lambda: (0, 0)),
        ],
        out_specs=[
            pl.BlockSpec((B, E), lambda: (0, 0)),
            pl.BlockSpec((B, K), lambda: (0, 0)),
        ],
        out_shape=[
            jax.ShapeDtypeStruct((B, E), jnp.float32),
            jax.ShapeDtypeStruct((B, K), jnp.int32),
        ],
        interpret=interpret,
    )(combined, W1, b1.reshape(1, C), W2, b2.reshape(1, E))
    return rw.reshape(B, E, 1, 1), idx.reshape(B, K, 1, 1)


def kernel(x, W1, b1, W2, b2):
    return _run(x, W1, b1, W2, b2)
